# Initial kernel scaffold; baseline (speedup 1.0000x reference)
import jax
import jax.numpy as jnp
from jax.experimental import pallas as pl


def kernel(input_ids, word_emb, pos_emb, ln_weight, ln_bias):
    raise NotImplementedError("write your pallas kernel here")


# speedup vs baseline: 2.1899x; 2.1899x over previous
import jax
import jax.numpy as jnp
from jax.experimental import pallas as pl

VOCAB = 511
DIM = 10
MAX_POS = 16
EPS = 1e-12
B = 16384
L = 16
VPAD = 512

RBLK = 512


def _embed_ln_body(ids_ref, w_ref, p_ref, g_ref, b_ref, out_ref):
    ids = ids_ref[0]
    ohT = (jnp.broadcast_to(ids, (VPAD, RBLK)) ==
           jax.lax.broadcasted_iota(jnp.int32, (VPAD, RBLK), 0)
           ).astype(jnp.float32)
    emb = jax.lax.dot_general(
        ohT, w_ref[...], (((0,), (0,)), ((), ())),
        preferred_element_type=jnp.float32)
    emb = emb + p_ref[...]
    mean = jnp.mean(emb, axis=-1, keepdims=True)
    var = jnp.mean(jnp.square(emb - mean), axis=-1, keepdims=True)
    xhat = (emb - mean) * jax.lax.rsqrt(var + EPS)
    out_ref[...] = xhat * g_ref[...] + b_ref[...]


def kernel(input_ids, word_emb, pos_emb, ln_weight, ln_bias):
    grid = (B * L) // RBLK
    ids3 = input_ids.reshape(grid, 1, RBLK)
    w_pad = jnp.zeros((VPAD, DIM), jnp.float32).at[:VOCAB].set(word_emb)
    pos_tiled = jnp.tile(pos_emb, (RBLK // L, 1))
    g2 = ln_weight.reshape(1, DIM)
    b2 = ln_bias.reshape(1, DIM)
    out = pl.pallas_call(
        _embed_ln_body,
        grid=(grid,),
        in_specs=[
            pl.BlockSpec((1, 1, RBLK), lambda i: (i, 0, 0)),
            pl.BlockSpec((VPAD, DIM), lambda i: (0, 0)),
            pl.BlockSpec((RBLK, DIM), lambda i: (0, 0)),
            pl.BlockSpec((1, DIM), lambda i: (0, 0)),
            pl.BlockSpec((1, DIM), lambda i: (0, 0)),
        ],
        out_specs=pl.BlockSpec((RBLK, DIM), lambda i: (i, 0)),
        out_shape=jax.ShapeDtypeStruct((B * L, DIM), jnp.float32),
    )(ids3, w_pad, pos_tiled, g2, b2)
    return out.reshape(B, L, DIM)


# speedup vs baseline: 3.1990x; 1.4608x over previous
import functools
import jax
import jax.numpy as jnp
from jax import lax
from jax.experimental import pallas as pl
from jax.experimental.pallas import tpu as pltpu
from jax.experimental.pallas import tpu_sc as plsc

VOCAB = 511
DIM = 10
MAX_POS = 16
EPS = 1e-12
B = 16384
L = 16
VPAD = 512
TROWS = VPAD * L

NC = 2
NS = 16
NW = NC * NS
ROWS = B * L
RPW = ROWS // NW
CHUNK = 128
NCH = RPW // CHUNK
FIRE = 16


DPAD = 16


def _table_ln_body(x_ref, g_ref, b_ref, out_ref):
    x = x_ref[...]
    mean = jnp.sum(x, axis=-1, keepdims=True) * (1.0 / DIM)
    var = jnp.sum(jnp.square(x), axis=-1, keepdims=True) * (1.0 / DIM) - jnp.square(mean)
    xhat = (x - mean) * lax.rsqrt(var + EPS)
    out_ref[...] = xhat * g_ref[...] + b_ref[...]


def _make_table(word_emb, pos_emb, ln_weight, ln_bias):
    w_pad = jnp.zeros((VPAD, DPAD), jnp.float32).at[:VOCAB, :DIM].set(word_emb)
    p_pad = jnp.zeros((L, DPAD), jnp.float32).at[:, :DIM].set(pos_emb)
    g_pad = jnp.ones((1, DPAD), jnp.float32).at[:, :DIM].set(ln_weight.reshape(1, DIM))
    b_pad = jnp.zeros((1, DPAD), jnp.float32).at[:, :DIM].set(ln_bias.reshape(1, DIM))
    x = jnp.repeat(w_pad, L, axis=0) + jnp.tile(p_pad, (VPAD, 1))
    return pl.pallas_call(
        _table_ln_body,
        in_specs=[
            pl.BlockSpec((TROWS, DPAD), lambda: (0, 0)),
            pl.BlockSpec((1, DPAD), lambda: (0, 0)),
            pl.BlockSpec((1, DPAD), lambda: (0, 0)),
        ],
        out_specs=pl.BlockSpec((TROWS, DPAD), lambda: (0, 0)),
        out_shape=jax.ShapeDtypeStruct((TROWS, DPAD), jnp.float32),
    )(x, g_pad, b_pad)


HALF = RPW // 2
HCH = NCH // 2


def _sc_gather_body(tab_hbm, ids_hbm, out_hbm, idx_v, rows_v, pack_v, sem):
    wid = lax.axis_index("s") * NC + lax.axis_index("c")
    base = wid * RPW
    pltpu.sync_copy(ids_hbm.at[wid], idx_v)
    liota = lax.iota(jnp.int32, 16)

    def idx_body(j, carry):
        for k in range(CHUNK // 16):
            sl = pl.ds(k * 16, 16)
            idx_v[j, sl] = idx_v[j, sl] * L + liota
        return carry

    lax.fori_loop(0, NCH, idx_body, 0)

    dimsplat = jnp.full((16,), DIM, jnp.int32)
    pats = []
    for j in range(DIM):
        p = liota + (j * 16)
        rvec = lax.div(p, dimsplat)
        pats.append((rvec, p - rvec * DIM))

    for h in range(2):
        for g in range(HCH // FIRE):
            descs = []
            for t in range(FIRE):
                j = h * HCH + g * FIRE + t
                descs.append(pltpu.async_copy(
                    tab_hbm.at[idx_v.at[j]],
                    rows_v.at[pl.ds((g * FIRE + t) * CHUNK, CHUNK), :],
                    sem))
            for d in descs:
                d.wait()

        def pack_body(grp, carry):
            for j in range(DIM):
                rvec, cvec = pats[j]
                v = plsc.load_gather(rows_v, [grp * 16 + rvec, cvec])
                pack_v[pl.ds(grp * (16 * DIM) + j * 16, 16)] = v
            return carry

        lax.fori_loop(0, HALF // 16, pack_body, 0)
        pltpu.sync_copy(pack_v,
                        out_hbm.at[pl.ds((base + h * HALF) * DIM, HALF * DIM)])


_sc_gather = functools.partial(
    pl.kernel,
    out_type=jax.ShapeDtypeStruct((ROWS * DIM,), jnp.float32),
    mesh=plsc.VectorSubcoreMesh(core_axis_name="c", subcore_axis_name="s"),
    scratch_types=[
        pltpu.VMEM((NCH, CHUNK), jnp.int32),
        pltpu.VMEM((HALF, DPAD), jnp.float32),
        pltpu.VMEM((HALF * DIM,), jnp.float32),
        pltpu.SemaphoreType.DMA,
    ],
    compiler_params=pltpu.CompilerParams(
        use_tc_tiling_on_sc=False, needs_layout_passes=False),
)(_sc_gather_body)


def kernel(input_ids, word_emb, pos_emb, ln_weight, ln_bias):
    table = _make_table(word_emb, pos_emb, ln_weight, ln_bias)
    ids3 = input_ids.reshape(NW, NCH, CHUNK)
    out = _sc_gather(table, ids3)
    return out.reshape(B, L, DIM)


# speedup vs baseline: 5.0947x; 1.5926x over previous
import functools
import jax
import jax.numpy as jnp
from jax import lax
from jax.experimental import pallas as pl
from jax.experimental.pallas import tpu as pltpu
from jax.experimental.pallas import tpu_sc as plsc

VOCAB = 511
DIM = 10
MAX_POS = 16
EPS = 1e-12
B = 16384
L = 16
VPAD = 512
TROWS = VPAD * L
TWORDS = TROWS * DIM

NC = 2
NS = 16
NW = NC * NS
ROWS = B * L
RPW = ROWS // NW
BPW = B // NW
IDC = 32
QROWS = 128
NQ = RPW // QROWS


def _table_ln_body(x_ref, g_ref, b_ref, out_ref):
    x = x_ref[...]
    mean = jnp.mean(x, axis=-1, keepdims=True)
    var = jnp.mean(jnp.square(x - mean), axis=-1, keepdims=True)
    xhat = (x - mean) * lax.rsqrt(var + EPS)
    out_ref[...] = xhat * g_ref[...] + b_ref[...]


def _make_table(word_emb, pos_emb, ln_weight, ln_bias):
    w_pad = jnp.zeros((VPAD, DIM), jnp.float32).at[:VOCAB].set(word_emb)
    x = jnp.repeat(w_pad, L, axis=0) + jnp.tile(pos_emb, (VPAD, 1))
    return pl.pallas_call(
        _table_ln_body,
        in_specs=[
            pl.BlockSpec((TROWS, DIM), lambda: (0, 0)),
            pl.BlockSpec((1, DIM), lambda: (0, 0)),
            pl.BlockSpec((1, DIM), lambda: (0, 0)),
        ],
        out_specs=pl.BlockSpec((TROWS, DIM), lambda: (0, 0)),
        out_shape=jax.ShapeDtypeStruct((TROWS, DIM), jnp.float32),
    )(x, ln_weight.reshape(1, DIM), ln_bias.reshape(1, DIM))


def _sc_gather_body(tab_hbm, ids_hbm, out_hbm, tab_v, ids_v, idx_v, buf_v,
                    sem_t, sem_i, sem_o):
    wid = lax.axis_index("s") * NC + lax.axis_index("c")
    base = wid * RPW
    bbase = wid * BPW
    tdesc = pltpu.async_copy(tab_hbm, tab_v, sem_t)
    liota = lax.iota(jnp.int32, 16)
    liota10 = liota * DIM

    for q in range(BPW // IDC):
        pltpu.async_copy(
            ids_hbm.at[pl.ds(bbase + q * IDC, IDC), :], ids_v, sem_i).wait()

        def fidx_body(j, carry):
            idx_v[pl.ds((q * IDC + j) * 16, 16)] = \
                ids_v[j, :] * (L * DIM) + liota10
            return carry

        lax.fori_loop(0, IDC, fidx_body, 0)

    dimsplat = jnp.full((16,), DIM, jnp.int32)
    pats = []
    for j in range(DIM):
        p = liota + (j * 16)
        rvec = lax.div(p, dimsplat)
        pats.append((rvec, p - rvec * DIM))

    tdesc.wait()
    descs = {}
    for h in range(NQ):
        par = h & 1
        if h >= 2:
            descs[h - 2].wait()

        def pack_body(g, carry):
            r0 = h * QROWS + g * 16
            for j in range(DIM):
                rvec, cvec = pats[j]
                wvec = plsc.load_gather(idx_v, [r0 + rvec]) + cvec
                v = plsc.load_gather(tab_v, [wvec])
                plsc.store_scatter(buf_v.at[par], [g * 16 + rvec, cvec], v)
            return carry

        lax.fori_loop(0, QROWS // 16, pack_body, 0)
        descs[h] = pltpu.async_copy(
            buf_v.at[par],
            out_hbm.at[pl.ds(base + h * QROWS, QROWS), :], sem_o)
    descs[NQ - 2].wait()
    descs[NQ - 1].wait()


_sc_gather = functools.partial(
    pl.kernel,
    out_type=jax.ShapeDtypeStruct((ROWS, DIM), jnp.float32),
    mesh=plsc.VectorSubcoreMesh(core_axis_name="c", subcore_axis_name="s"),
    scratch_types=[
        pltpu.VMEM((TWORDS,), jnp.float32),
        pltpu.VMEM((IDC, L), jnp.int32),
        pltpu.VMEM((RPW,), jnp.int32),
        pltpu.VMEM((2, QROWS, DIM), jnp.float32),
        pltpu.SemaphoreType.DMA,
        pltpu.SemaphoreType.DMA,
        pltpu.SemaphoreType.DMA,
    ],
    compiler_params=pltpu.CompilerParams(
        use_tc_tiling_on_sc=True, needs_layout_passes=False),
)(_sc_gather_body)


def kernel(input_ids, word_emb, pos_emb, ln_weight, ln_bias):
    table = _make_table(word_emb, pos_emb, ln_weight, ln_bias)
    out = _sc_gather(table.reshape(TWORDS), input_ids)
    return out.reshape(B, L, DIM)
